# Initial kernel scaffold; baseline (speedup 1.0000x reference)
import jax
import jax.numpy as jnp
from jax.experimental import pallas as pl


def kernel(s_feats, q_points, s_points, neighbor_indices):
    raise NotImplementedError("write your pallas kernel here")


# speedup vs baseline: 3.1547x; 3.1547x over previous
import functools

import jax
import jax.numpy as jnp
from jax import lax
from jax.experimental import pallas as pl
from jax.experimental.pallas import tpu as pltpu
from jax.experimental.pallas import tpu_sc as plsc

K = 8
EPS = 1e-8
S = 12500
Q = 50000
D = 256

S_PAD = 12512
NW = 32
PER_TILE = 1568
QP = PER_TILE * NW
PAIRS = PER_TILE // 2
HALF = PAIRS // 2
OB_ROWS = 32


def _tile_body(sfeats_hbm, idx_hbm, qpt_hbm, spt_hbm, out_hbm,
               sp_v, q_v, idx_v, rows0, rows1, ob,
               sem_g0, sem_g1, sem_o):
    nc = 2
    wid = lax.axis_index("s") * nc + lax.axis_index("c")
    qbase = wid * PER_TILE

    pltpu.sync_copy(spt_hbm, sp_v)
    pltpu.sync_copy(qpt_hbm.at[pl.ds(qbase, PER_TILE), :], q_v)
    pltpu.sync_copy(idx_hbm.at[pl.ds(qbase * K, PER_TILE * K)], idx_v)

    iota = lax.iota(jnp.int32, 16)
    row0 = jnp.zeros((16,), jnp.int32)
    row1 = jnp.full((16,), 1, jnp.int32)
    row2 = jnp.full((16,), 2, jnp.int32)

    def idx_vec(g):
        return idx_v[pl.ds(g * 16, 16)]

    def gather_desc(g, rows_ref, sem):
        return pltpu.make_async_copy(sfeats_hbm.at[idx_vec(g)], rows_ref, sem)

    def out_desc(blk):
        return pltpu.make_async_copy(
            ob, out_hbm.at[pl.ds(qbase + blk * OB_ROWS, OB_ROWS), :], sem_o)

    def compute_pair(g, rows, rb):
        lq = g * 2
        idxv = idx_vec(g)
        px = plsc.load_gather(sp_v, [row0, idxv])
        py = plsc.load_gather(sp_v, [row1, idxv])
        pz = plsc.load_gather(sp_v, [row2, idxv])
        qsel = jnp.where(iota >= 8, lq + 1, lq).astype(jnp.int32)
        qx = plsc.load_gather(q_v, [qsel, row0])
        qy = plsc.load_gather(q_v, [qsel, row1])
        qz = plsc.load_gather(q_v, [qsel, row2])
        dx = qx - px
        dy = qy - py
        dz = qz - pz
        d2 = dx * dx + dy * dy + dz * dz
        m = jnp.where(idxv != S, 1.0, 0.0).astype(jnp.float32)
        w = m / (d2 + EPS)
        sa = jnp.sum(jnp.where(iota < 8, w, 0.0))
        st = jnp.sum(w)
        tot = jnp.where(iota < 8, sa, st - sa)
        wn = w / (tot + EPS)
        wb = [jnp.sum(jnp.where(iota == k, wn, 0.0)) for k in range(16)]
        for ch in range(D // 16):
            sl = pl.ds(ch * 16, 16)
            a0 = rows[0, sl] * wb[0]
            a1 = rows[8, sl] * wb[8]
            for k in range(1, 8):
                a0 = a0 + rows[k, sl] * wb[k]
                a1 = a1 + rows[8 + k, sl] * wb[8 + k]
            ob[rb, sl] = a0
            ob[rb + 1, sl] = a1

    gather_desc(0, rows0, sem_g0).start()
    gather_desc(1, rows1, sem_g1).start()

    def body(j, carry):
        g0 = j * 2
        g1 = g0 + 1
        g0n = jnp.minimum(g0 + 2, PAIRS - 1)
        g1n = jnp.minimum(g1 + 2, PAIRS - 1)
        rb = (j % 8) * 4

        @pl.when(jnp.logical_and(j % 8 == 0, j >= 8))
        def _():
            out_desc(j // 8 - 1).wait()

        gather_desc(g0, rows0, sem_g0).wait()
        compute_pair(g0, rows0, rb)
        gather_desc(g0n, rows0, sem_g0).start()

        gather_desc(g1, rows1, sem_g1).wait()
        compute_pair(g1, rows1, rb + 2)
        gather_desc(g1n, rows1, sem_g1).start()

        @pl.when(j % 8 == 7)
        def _():
            out_desc(j // 8).start()

        return carry

    lax.fori_loop(0, HALF, body, 0)

    gather_desc(PAIRS - 1, rows0, sem_g0).wait()
    gather_desc(PAIRS - 1, rows1, sem_g1).wait()
    out_desc(HALF // 8 - 1).wait()


def kernel(s_feats, q_points, s_points, neighbor_indices):
    s_feats = s_feats.astype(jnp.float32)
    q_points = q_points.astype(jnp.float32)
    s_points = s_points.astype(jnp.float32)
    idx = neighbor_indices.astype(jnp.int32)

    sfeats_p = jnp.zeros((S_PAD, D), jnp.float32).at[:S].set(s_feats)
    spt = jnp.zeros((3, S_PAD), jnp.float32).at[:, :S].set(s_points.T)
    qpt = jnp.zeros((QP, 3), jnp.float32).at[:Q].set(q_points)
    idx_p = jnp.full((QP, K), S, jnp.int32).at[:Q].set(idx[:, :K])
    idx_flat = idx_p.reshape(QP * K)

    mesh = plsc.VectorSubcoreMesh(core_axis_name="c", subcore_axis_name="s")
    run = pl.kernel(
        _tile_body,
        mesh=mesh,
        compiler_params=pltpu.CompilerParams(
            use_tc_tiling_on_sc=False, needs_layout_passes=False),
        out_type=jax.ShapeDtypeStruct((QP, D), jnp.float32),
        scratch_types=[
            pltpu.VMEM((3, S_PAD), jnp.float32),
            pltpu.VMEM((PER_TILE, 3), jnp.float32),
            pltpu.VMEM((PER_TILE * K,), jnp.int32),
            pltpu.VMEM((16, D), jnp.float32),
            pltpu.VMEM((16, D), jnp.float32),
            pltpu.VMEM((OB_ROWS, D), jnp.float32),
            pltpu.SemaphoreType.DMA,
            pltpu.SemaphoreType.DMA,
            pltpu.SemaphoreType.DMA,
        ],
    )
    out = run(sfeats_p, idx_flat, qpt, spt)
    return out[:Q]
